# baseline (speedup 1.0000x reference)
import functools

import jax
import jax.numpy as jnp
from jax import lax
from jax.experimental import pallas as pl
from jax.experimental.pallas import tpu as pltpu
from jax.experimental.pallas import tpu_sc as plsc

D = 64
B = 1024
S = 200
NC, NS = 2, 16
NW = NC * NS
NBB = B // 128
NCB = S * NBB // NW
LANES = 16
KD = D // LANES

_mesh = plsc.VectorSubcoreMesh(core_axis_name="c", subcore_axis_name="s")


@functools.partial(
    pl.kernel,
    out_type=jax.ShapeDtypeStruct((B, S, D), jnp.float32),
    mesh=_mesh,
    scratch_types=[
        pltpu.VMEM((S, 128), jnp.int32),
        pltpu.VMEM((3, 128, D), jnp.float32),
        pltpu.VMEM((2, 128, D), jnp.float32),
        pltpu.VMEM((S, D), jnp.float32),
        pltpu.SemaphoreType.DMA((3,)),
        pltpu.SemaphoreType.DMA((2,)),
    ],
    compiler_params=pltpu.CompilerParams(use_tc_tiling_on_sc=False),
)
def _emb_lookup(ids_hbm, table_hbm, pos_hbm, out_hbm, ids_v, gbuf,
                tstage, pos_v, gsem, ssem):
    wid = lax.axis_index("s") * NC + lax.axis_index("c")
    bb = lax.rem(wid, NBB)
    sq = wid // NBB
    pltpu.sync_copy(pos_hbm.at[pl.ds(0, S)], pos_v)
    pltpu.sync_copy(ids_hbm.at[:, pl.ds(bb * 128, 128)], ids_v)

    def fire_gather(k):
        b3 = lax.rem(k, 3)
        pltpu.async_copy(table_hbm.at[ids_v.at[sq + 4 * k]],
                         gbuf.at[b3], gsem.at[b3])

    def wait_gather(k):
        b3 = lax.rem(k, 3)
        pltpu.make_async_copy(table_hbm.at[ids_v.at[sq + 4 * k]],
                              gbuf.at[b3], gsem.at[b3]).wait()

    def fire_out(k):
        tb = lax.rem(k, 2)
        s = sq + 4 * k
        pltpu.async_copy(tstage.at[tb],
                         out_hbm.at[pl.ds(bb * 128, 128), s], ssem.at[tb])

    def wait_out(k):
        tb = lax.rem(k, 2)
        s = sq + 4 * k
        pltpu.make_async_copy(tstage.at[tb],
                              out_hbm.at[pl.ds(bb * 128, 128), s],
                              ssem.at[tb]).wait()

    fire_gather(0)
    fire_gather(1)

    def body(k, carry):
        b3 = lax.rem(k, 3)
        tb = lax.rem(k, 2)
        s = sq + 4 * k

        @pl.when(k < NCB - 2)
        def _():
            fire_gather(k + 2)

        wait_gather(k)

        @pl.when(k >= 2)
        def _():
            wait_out(k - 2)

        pj = [pos_v[s, pl.ds(16 * j, 16)] for j in range(KD)]
        for r in range(128):
            for j in range(KD):
                sl = pl.ds(16 * j, 16)
                tstage[tb, r, sl] = gbuf[b3, r, sl] + pj[j]

        fire_out(k)
        return carry

    lax.fori_loop(0, NCB, body, 0)
    wait_out(NCB - 2)
    wait_out(NCB - 1)


def kernel(token_ids, token_table, pos_table):
    ids_t = token_ids.T.astype(jnp.int32)
    table2 = jnp.concatenate(
        [token_table[0::2], token_table[1::2]], axis=1).reshape(1000000, D)
    return _emb_lookup(ids_t, table2, pos_table)


# speedup vs baseline: 9.4610x; 9.4610x over previous
import functools

import jax
import jax.numpy as jnp
from jax import lax
from jax.experimental import pallas as pl
from jax.experimental.pallas import tpu as pltpu
from jax.experimental.pallas import tpu_sc as plsc

D = 64
B = 1024
S = 200
V = 1000000
NC, NS = 2, 16
NW = NC * NS
BPW = B // NW
LANES = 16
KD = D // LANES

TRV = 1024
TRG = (V + TRV - 1) // TRV

_mesh = plsc.VectorSubcoreMesh(core_axis_name="c", subcore_axis_name="s")


def _tr_body(x_ref, y_ref):
    a = x_ref[...].T.reshape(TRV // 2, 2, D)
    y_ref[:, 0:D] = a[:, 0, :]
    y_ref[:, D:128] = a[:, 1, :]


def _transpose_table(tT):
    return pl.pallas_call(
        _tr_body,
        grid=(TRG,),
        in_specs=[pl.BlockSpec((D, TRV), lambda i: (0, i))],
        out_specs=pl.BlockSpec((TRV // 2, 128), lambda i: (i, 0)),
        out_shape=jax.ShapeDtypeStruct((V // 2, 128), jnp.float32),
    )(tT)


@functools.partial(
    pl.kernel,
    out_type=jax.ShapeDtypeStruct((B * S, D), jnp.float32),
    mesh=_mesh,
    scratch_types=[
        pltpu.VMEM((BPW, S), jnp.int32),
        pltpu.VMEM((3, S, D), jnp.float32),
        pltpu.VMEM((2, S, D), jnp.float32),
        pltpu.VMEM((S, D), jnp.float32),
        pltpu.SemaphoreType.DMA((3,)),
        pltpu.SemaphoreType.DMA((2,)),
    ],
    compiler_params=pltpu.CompilerParams(use_tc_tiling_on_sc=False),
)
def _emb_lookup(ids_hbm, table_hbm, pos_hbm, out_hbm, ids_v, gbuf,
                tstage, pos_v, gsem, ssem):
    wid = lax.axis_index("s") * NC + lax.axis_index("c")
    b0 = wid * BPW
    pltpu.sync_copy(pos_hbm.at[pl.ds(0, S)], pos_v)
    pltpu.sync_copy(ids_hbm.at[pl.ds(b0, BPW)], ids_v)

    def fire_gather(k):
        b3 = lax.rem(k, 3)
        pltpu.async_copy(table_hbm.at[ids_v.at[k, pl.ds(0, 128)]],
                         gbuf.at[b3, pl.ds(0, 128)], gsem.at[b3])
        pltpu.async_copy(table_hbm.at[ids_v.at[k, pl.ds(128, S - 128)]],
                         gbuf.at[b3, pl.ds(128, S - 128)], gsem.at[b3])

    def wait_gather(k):
        b3 = lax.rem(k, 3)
        pltpu.make_async_copy(table_hbm.at[ids_v.at[k, pl.ds(0, 128)]],
                              gbuf.at[b3, pl.ds(0, 128)],
                              gsem.at[b3]).wait()
        pltpu.make_async_copy(table_hbm.at[ids_v.at[k, pl.ds(128, S - 128)]],
                              gbuf.at[b3, pl.ds(128, S - 128)],
                              gsem.at[b3]).wait()

    def fire_out(k):
        tb = lax.rem(k, 2)
        pltpu.async_copy(tstage.at[tb],
                         out_hbm.at[pl.ds((b0 + k) * S, S)], ssem.at[tb])

    def wait_out(k):
        tb = lax.rem(k, 2)
        pltpu.make_async_copy(tstage.at[tb],
                              out_hbm.at[pl.ds((b0 + k) * S, S)],
                              ssem.at[tb]).wait()

    fire_gather(0)
    fire_gather(1)

    def body(k, carry):
        b3 = lax.rem(k, 3)
        tb = lax.rem(k, 2)

        @pl.when(k < BPW - 2)
        def _():
            fire_gather(k + 2)

        wait_gather(k)

        @pl.when(k >= 2)
        def _():
            wait_out(k - 2)

        @plsc.parallel_loop(0, S, unroll=4)
        def _(s):
            for j in range(KD):
                sl = pl.ds(16 * j, 16)
                tstage[tb, s, sl] = gbuf[b3, s, sl] + pos_v[s, sl]

        fire_out(k)
        return carry

    lax.fori_loop(0, BPW, body, 0)
    wait_out(BPW - 2)
    wait_out(BPW - 1)


def kernel(token_ids, token_table, pos_table):
    table2 = _transpose_table(token_table.T)
    table_lin = table2.reshape(V, D)
    out = _emb_lookup(token_ids.astype(jnp.int32), table_lin, pos_table)
    return out.reshape(B, S, D)


# speedup vs baseline: 11.7555x; 1.2425x over previous
import functools

import jax
import jax.numpy as jnp
from jax import lax
from jax.experimental import pallas as pl
from jax.experimental.pallas import tpu as pltpu
from jax.experimental.pallas import tpu_sc as plsc

D = 64
B = 1024
S = 200
V = 1000000
NC, NS = 2, 16
NW = NC * NS
BPW = B // NW
LANES = 16
KD = D // LANES

TRV = 1024
TRG = (V + TRV - 1) // TRV

_mesh = plsc.VectorSubcoreMesh(core_axis_name="c", subcore_axis_name="s")


def _tr_body(x_ref, y_ref):
    a = x_ref[...].T.reshape(TRV // 2, 2, D)
    y_ref[:, 0:D] = a[:, 0, :]
    y_ref[:, D:128] = a[:, 1, :]


def _transpose_table(tT):
    return pl.pallas_call(
        _tr_body,
        grid=(TRG,),
        in_specs=[pl.BlockSpec((D, TRV), lambda i: (0, i))],
        out_specs=pl.BlockSpec((TRV // 2, 128), lambda i: (i, 0)),
        out_shape=jax.ShapeDtypeStruct((V // 2, 128), jnp.float32),
    )(tT)


@functools.partial(
    pl.kernel,
    out_type=jax.ShapeDtypeStruct((B * S, D), jnp.float32),
    mesh=_mesh,
    scratch_types=[
        pltpu.VMEM((BPW, S), jnp.int32),
        pltpu.VMEM((3, S, D), jnp.float32),
        pltpu.VMEM((2, S, D), jnp.float32),
        pltpu.VMEM((S, D), jnp.float32),
        pltpu.SemaphoreType.DMA((3,)),
        pltpu.SemaphoreType.DMA((2,)),
    ],
    compiler_params=pltpu.CompilerParams(use_tc_tiling_on_sc=False),
)
def _emb_lookup(ids_hbm, table_hbm, pos_hbm, out_hbm, ids_v, gbuf,
                tstage, pos_v, gsem, ssem):
    wid = lax.axis_index("s") * NC + lax.axis_index("c")
    b0 = wid * BPW
    pltpu.sync_copy(pos_hbm.at[pl.ds(0, S)], pos_v)
    pltpu.sync_copy(ids_hbm.at[pl.ds(b0, BPW)], ids_v)

    def fire_gather(k):
        b3 = lax.rem(k, 3)
        pltpu.async_copy(table_hbm.at[ids_v.at[k, pl.ds(0, 128)]],
                         gbuf.at[b3, pl.ds(0, 128)], gsem.at[b3])
        pltpu.async_copy(table_hbm.at[ids_v.at[k, pl.ds(128, S - 128)]],
                         gbuf.at[b3, pl.ds(128, S - 128)], gsem.at[b3])

    def wait_gather(k):
        b3 = lax.rem(k, 3)
        pltpu.make_async_copy(table_hbm.at[ids_v.at[k, pl.ds(0, 128)]],
                              gbuf.at[b3, pl.ds(0, 128)],
                              gsem.at[b3]).wait()
        pltpu.make_async_copy(table_hbm.at[ids_v.at[k, pl.ds(128, S - 128)]],
                              gbuf.at[b3, pl.ds(128, S - 128)],
                              gsem.at[b3]).wait()

    def fire_out(k):
        tb = lax.rem(k, 2)
        pltpu.async_copy(tstage.at[tb],
                         out_hbm.at[pl.ds((b0 + k) * S, S)], ssem.at[tb])

    def wait_out(k):
        tb = lax.rem(k, 2)
        pltpu.make_async_copy(tstage.at[tb],
                              out_hbm.at[pl.ds((b0 + k) * S, S)],
                              ssem.at[tb]).wait()

    fire_gather(0)
    fire_gather(1)

    def body(k, carry):
        b3 = lax.rem(k, 3)
        tb = lax.rem(k, 2)

        @pl.when(k < BPW - 2)
        def _():
            fire_gather(k + 2)

        wait_gather(k)

        @pl.when(k >= 2)
        def _():
            wait_out(k - 2)

        @plsc.parallel_loop(0, S, unroll=4)
        def _(s):
            for j in range(KD):
                sl = pl.ds(16 * j, 16)
                tstage[tb, s, sl] = gbuf[b3, s, sl] + pos_v[s, sl]

        fire_out(k)
        return carry

    lax.fori_loop(0, BPW, body, 0)
    wait_out(BPW - 2)
    wait_out(BPW - 1)


def kernel(token_ids, token_table, pos_table):
    table2 = lax.optimization_barrier(token_table.reshape(V // 2, 128))
    table_lin = table2.reshape(V, D)
    out = _emb_lookup(token_ids.astype(jnp.int32), table_lin, pos_table)
    return out.reshape(B, S, D)
